# baseline (speedup 1.0000x reference)
import functools

import jax
import jax.numpy as jnp
from jax import lax
from jax.experimental import pallas as pl
from jax.experimental.pallas import tpu as pltpu
from jax.experimental.pallas import tpu_sc as plsc

B = 16384
D = 64
CHUNK = 128
NC, NS = 2, 16
NW = NC * NS
B_PER_W = B // NW
K_PER_W = B_PER_W // CHUNK


def _sc_gather_body(ptab_hbm, atab_hbm, pidx_hbm, aidx_hbm,
                    pout_hbm, aout_hbm,
                    pidx_v, aidx_v, prows_v, arows_v, sem):
    wid = lax.axis_index("s") * NC + lax.axis_index("c")
    row0 = wid * K_PER_W
    pltpu.sync_copy(pidx_hbm.at[pl.ds(row0, K_PER_W)], pidx_v)
    pltpu.sync_copy(aidx_hbm.at[pl.ds(row0, K_PER_W)], aidx_v)
    copies = []
    for j in range(K_PER_W):
        copies.append(pltpu.async_copy(ptab_hbm.at[pidx_v.at[j]],
                                       prows_v.at[j], sem))
        copies.append(pltpu.async_copy(atab_hbm.at[aidx_v.at[j]],
                                       arows_v.at[j], sem))
    for c in copies:
        c.wait()
    pltpu.sync_copy(prows_v, pout_hbm.at[pl.ds(row0, K_PER_W)])
    pltpu.sync_copy(arows_v, aout_hbm.at[pl.ds(row0, K_PER_W)])


@jax.jit
def _sc_gather(product_table, action_type_table, product_ids, action_types):
    pidx = product_ids.reshape(B // CHUNK, CHUNK)
    aidx = action_types.reshape(B // CHUNK, CHUNK)
    mesh = plsc.VectorSubcoreMesh(core_axis_name="c", subcore_axis_name="s")
    out_t = (jax.ShapeDtypeStruct((B // CHUNK, CHUNK, D), jnp.float32),
             jax.ShapeDtypeStruct((B // CHUNK, CHUNK, D), jnp.float32))
    fn = pl.kernel(
        _sc_gather_body,
        mesh=mesh,
        out_type=out_t,
        compiler_params=pltpu.CompilerParams(use_tc_tiling_on_sc=False),
        scratch_types=[
            pltpu.VMEM((K_PER_W, CHUNK), jnp.int32),
            pltpu.VMEM((K_PER_W, CHUNK), jnp.int32),
            pltpu.VMEM((K_PER_W, CHUNK, D), jnp.float32),
            pltpu.VMEM((K_PER_W, CHUNK, D), jnp.float32),
            pltpu.SemaphoreType.DMA,
        ],
    )
    pe, ae = fn(product_table, action_type_table, pidx, aidx)
    return pe.reshape(B, D), ae.reshape(B, D)


def _dense_body(ae_ref, pe_ref, tf_ref, cf_ref,
                tw_ref, tb_ref, cw_ref, cb_ref,
                wa_ref, wp_ref, wt_ref, wc_ref, fb_ref, out_ref):
    f32 = jnp.float32
    cdims = (((1,), (1,)), ((), ()))
    t_emb = lax.dot_general(tf_ref[...], tw_ref[...], cdims,
                            preferred_element_type=f32) + tb_ref[...]
    c_emb = lax.dot_general(cf_ref[...], cw_ref[...], cdims,
                            preferred_element_type=f32) + cb_ref[...]
    mm = (((1,), (0,)), ((), ()))
    acc = lax.dot_general(ae_ref[...], wa_ref[...], mm, preferred_element_type=f32)
    acc += lax.dot_general(pe_ref[...], wp_ref[...], mm, preferred_element_type=f32)
    acc += lax.dot_general(t_emb, wt_ref[...], mm, preferred_element_type=f32)
    acc += lax.dot_general(c_emb, wc_ref[...], mm, preferred_element_type=f32)
    out_ref[...] = jnp.maximum(acc + fb_ref[...], 0.0)


@functools.partial(jax.jit, static_argnames=("blk",))
def _tc_dense(action_emb, product_emb, temporal_features, context_features,
              temporal_W, temporal_b, context_W, context_b,
              wa, wp, wt, wc, fb, blk=2048):
    grid = (B // blk,)
    row_spec = lambda d: pl.BlockSpec((blk, d), lambda i: (i, 0))
    full = lambda a: pl.BlockSpec(a.shape, lambda i: (0,) * a.ndim)
    return pl.pallas_call(
        _dense_body,
        grid=grid,
        in_specs=[
            row_spec(D), row_spec(D), row_spec(5), row_spec(10),
            full(temporal_W), full(temporal_b), full(context_W), full(context_b),
            full(wa), full(wp), full(wt), full(wc), full(fb),
        ],
        out_specs=pl.BlockSpec((blk, 128), lambda i: (i, 0)),
        out_shape=jax.ShapeDtypeStruct((B, 128), jnp.float32),
    )(action_emb, product_emb, temporal_features, context_features,
      temporal_W, temporal_b, context_W, context_b, wa, wp, wt, wc, fb)


def kernel(action_types, product_ids, temporal_features, context_features,
           action_type_table, product_table,
           temporal_W, temporal_b, context_W, context_b,
           fusion_W, fusion_b):
    pe, ae = _sc_gather(product_table, action_type_table,
                        product_ids, action_types)
    wa = fusion_W[:, 0:64].T
    wp = fusion_W[:, 64:128].T
    wt = fusion_W[:, 128:160].T
    wc = fusion_W[:, 160:192].T
    return _tc_dense(ae, pe, temporal_features, context_features,
                     temporal_W, temporal_b.reshape(1, 32),
                     context_W, context_b.reshape(1, 32),
                     wa, wp, wt, wc, fusion_b.reshape(1, 128))


# speedup vs baseline: 1.5757x; 1.5757x over previous
import functools

import jax
import jax.numpy as jnp
from jax import lax
from jax.experimental import pallas as pl
from jax.experimental.pallas import tpu as pltpu
from jax.experimental.pallas import tpu_sc as plsc

B = 16384
D = 64
CHUNK = 128
NC, NS = 2, 16
NW = NC * NS
B_PER_W = B // NW
K_PER_W = B_PER_W // CHUNK
NGRP = CHUNK // 16


DMA_BLOCK = 32


def _sc_gather_body(ptab_hbm, atab_hbm, pidx_hbm, aidx_hbm,
                    pe_hbm, ae_hbm,
                    pidx_v, aidx_v, rows_v, awide_v, sem, asem):
    wid = lax.axis_index("s") * NC + lax.axis_index("c")
    rowbase = wid * B_PER_W
    iota16 = lax.iota(jnp.int32, 16)
    pltpu.sync_copy(pidx_hbm.at[pl.ds(wid, 1)], pidx_v)
    pltpu.sync_copy(aidx_hbm.at[pl.ds(wid * K_PER_W, K_PER_W)], aidx_v)

    def fetch_block(b):
        base = b * DMA_BLOCK
        copies = []
        for g in range(DMA_BLOCK // 16):
            v = pidx_v[0, pl.ds(base + g * 16, 16)]
            for l in range(16):
                r = v[l]
                copies.append(pltpu.async_copy(
                    ptab_hbm.at[pl.ds(r, 1)],
                    rows_v.at[pl.ds(base + g * 16 + l, 1)], sem))
        for c in copies:
            c.wait()

    pl.loop(0, B_PER_W // DMA_BLOCK)(fetch_block)
    pltpu.sync_copy(rows_v, pe_hbm.at[pl.ds(rowbase, B_PER_W)])
    for j in range(K_PER_W):
        pltpu.async_copy(atab_hbm.at[aidx_v.at[j]], awide_v, asem).wait()
        pltpu.sync_copy(awide_v,
                        ae_hbm.at[pl.ds(rowbase + j * CHUNK, CHUNK)])


@jax.jit
def _sc_gather(product_table, action_type_table, product_ids, action_types):
    atab_p = jnp.pad(action_type_table, ((0, 0), (0, 64)))
    pidx = product_ids.reshape(NW, B_PER_W)
    aidx = action_types.reshape(B // CHUNK, CHUNK)
    mesh = plsc.VectorSubcoreMesh(core_axis_name="c", subcore_axis_name="s")
    out_t = (jax.ShapeDtypeStruct((B, D), jnp.float32),
             jax.ShapeDtypeStruct((B, 128), jnp.float32))
    fn = pl.kernel(
        _sc_gather_body,
        mesh=mesh,
        out_type=out_t,
        scratch_types=[
            pltpu.VMEM((1, B_PER_W), jnp.int32),
            pltpu.VMEM((K_PER_W, CHUNK), jnp.int32),
            pltpu.VMEM((B_PER_W, D), jnp.float32),
            pltpu.VMEM((CHUNK, 128), jnp.float32),
            pltpu.SemaphoreType.DMA,
            pltpu.SemaphoreType.DMA,
        ],
    )
    return fn(product_table, atab_p, pidx, aidx)


def _dense_body(ae_ref, pe_ref, tf_ref, cf_ref,
                tw_ref, tb_ref, cw_ref, cb_ref,
                wa_ref, wp_ref, wt_ref, wc_ref, fb_ref, out_ref):
    f32 = jnp.float32
    cdims = (((1,), (1,)), ((), ()))
    t_emb = lax.dot_general(tf_ref[...], tw_ref[...], cdims,
                            preferred_element_type=f32) + tb_ref[...]
    c_emb = lax.dot_general(cf_ref[...], cw_ref[...], cdims,
                            preferred_element_type=f32) + cb_ref[...]
    mm = (((1,), (0,)), ((), ()))
    acc = lax.dot_general(ae_ref[...], wa_ref[...], mm, preferred_element_type=f32)
    acc += lax.dot_general(pe_ref[...], wp_ref[...], mm, preferred_element_type=f32)
    acc += lax.dot_general(t_emb, wt_ref[...], mm, preferred_element_type=f32)
    acc += lax.dot_general(c_emb, wc_ref[...], mm, preferred_element_type=f32)
    out_ref[...] = jnp.maximum(acc + fb_ref[...], 0.0)


@functools.partial(jax.jit, static_argnames=("blk",))
def _tc_dense(action_emb, product_emb, temporal_features, context_features,
              temporal_W, temporal_b, context_W, context_b,
              wa, wp, wt, wc, fb, blk=2048):
    grid = (B // blk,)
    row_spec = lambda d: pl.BlockSpec((blk, d), lambda i: (i, 0))
    full = lambda a: pl.BlockSpec(a.shape, lambda i: (0,) * a.ndim)
    return pl.pallas_call(
        _dense_body,
        grid=grid,
        in_specs=[
            row_spec(128), row_spec(D), row_spec(5), row_spec(10),
            full(temporal_W), full(temporal_b), full(context_W), full(context_b),
            full(wa), full(wp), full(wt), full(wc), full(fb),
        ],
        out_specs=pl.BlockSpec((blk, 128), lambda i: (i, 0)),
        out_shape=jax.ShapeDtypeStruct((B, 128), jnp.float32),
    )(action_emb, product_emb, temporal_features, context_features,
      temporal_W, temporal_b, context_W, context_b, wa, wp, wt, wc, fb)


def kernel(action_types, product_ids, temporal_features, context_features,
           action_type_table, product_table,
           temporal_W, temporal_b, context_W, context_b,
           fusion_W, fusion_b):
    pe, ae_w = _sc_gather(product_table, action_type_table,
                          product_ids, action_types)
    wa_p = jnp.concatenate(
        [fusion_W[:, 0:64].T, jnp.zeros((64, 128), jnp.float32)], axis=0)
    wp = fusion_W[:, 64:128].T
    wt = fusion_W[:, 128:160].T
    wc = fusion_W[:, 160:192].T
    return _tc_dense(ae_w, pe, temporal_features, context_features,
                     temporal_W, temporal_b.reshape(1, 32),
                     context_W, context_b.reshape(1, 32),
                     wa_p, wp, wt, wc, fusion_b.reshape(1, 128))
